# baseline (speedup 1.0000x reference)
import jax
import jax.numpy as jnp
from jax import lax
from jax.experimental import pallas as pl
from jax.experimental.pallas import tpu as pltpu
from jax.experimental.pallas import tpu_sc as plsc

N_CENTER = 100000
D = 64
B = 16384
ALPHA = 0.9
A1 = ALPHA - 1.0

NC = 2
NS = 16
BT = B // NS
HALF = N_CENTER // NC
NCHUNK = 2
CHUNK = HALF // NCHUNK
TROWS = 1568
GBLK = 64
NJ = BT // GBLK
DBLK = 128
NDB = 13


def _body(feat_hbm, lab_hbm, ctr_hbm, out_hbm,
          gath_blk, feat_blk, lab_v, idx_v, ctr_blk, acc_blk,
          acc_sh):
    c = lax.axis_index("c")
    s = lax.axis_index("s")

    pltpu.sync_copy(lab_hbm.at[s], lab_v)

    tstart = jnp.minimum(s * TROWS, CHUNK - TROWS)

    for chunk in range(NCHUNK):
        base = c * HALF + chunk * CHUNK

        def fill_zero(i, _):
            gath_blk[i // 4, pl.ds((i % 4) * 16, 16)] = jnp.zeros(
                (16,), jnp.float32)
            return 0
        lax.fori_loop(0, GBLK * 4, fill_zero, 0)
        for b in range(NDB):
            rb = tstart + min(b * DBLK, TROWS - DBLK)
            pltpu.sync_copy(gath_blk, acc_sh.at[pl.ds(rb, GBLK)])
            pltpu.sync_copy(gath_blk, acc_sh.at[pl.ds(rb + GBLK, GBLK)])
        plsc.subcore_barrier()

        for j in range(NJ):
            pltpu.sync_copy(ctr_hbm.at[lab_v.at[j]], gath_blk)
            pltpu.sync_copy(feat_hbm.at[pl.ds(s * BT + j * GBLK, GBLK)],
                            feat_blk)

            def mkdiff(i, _):
                r = i // 4
                g = (i % 4) * 16
                gath_blk[r, pl.ds(g, 16)] = A1 * (
                    gath_blk[r, pl.ds(g, 16)] - feat_blk[r, pl.ds(g, 16)])
                return 0
            lax.fori_loop(0, GBLK * 4, mkdiff, 0)

            for k in range(GBLK // 16):
                v = lab_v[j, pl.ds(k * 16, 16)]
                rel = v - base
                inb = (rel >= 0) & (rel < CHUNK)
                idx_v[j, pl.ds(k * 16, 16)] = jnp.where(inb, rel, CHUNK)

            pltpu.sync_copy(gath_blk, acc_sh.at[idx_v.at[j]], add=True)
        plsc.subcore_barrier()

        for b in range(NDB):
            rb = tstart + min(b * DBLK, TROWS - DBLK)
            row = base + rb
            pltpu.sync_copy(ctr_hbm.at[pl.ds(row, DBLK)], ctr_blk)
            pltpu.sync_copy(acc_sh.at[pl.ds(rb, DBLK)], acc_blk)

            def combine(i, _):
                r = i // 4
                g = (i % 4) * 16
                ctr_blk[r, pl.ds(g, 16)] = (
                    ctr_blk[r, pl.ds(g, 16)] + acc_blk[r, pl.ds(g, 16)])
                return 0
            lax.fori_loop(0, DBLK * 4, combine, 0)

            pltpu.sync_copy(ctr_blk, out_hbm.at[pl.ds(row, DBLK)])

        plsc.subcore_barrier()


@jax.jit
def _run(features, labels, centers):
    mesh = plsc.VectorSubcoreMesh(core_axis_name="c", subcore_axis_name="s")
    lab3 = labels.reshape(NS, NJ, GBLK)
    return pl.kernel(
        _body,
        out_type=jax.ShapeDtypeStruct((N_CENTER, D), jnp.float32),
        mesh=mesh,
        compiler_params=pltpu.CompilerParams(use_tc_tiling_on_sc=False),
        scratch_types=[
            pltpu.VMEM((GBLK, D), jnp.float32),
            pltpu.VMEM((GBLK, D), jnp.float32),
            pltpu.VMEM((NJ, GBLK), jnp.int32),
            pltpu.VMEM((NJ, GBLK), jnp.int32),
            pltpu.VMEM((DBLK, D), jnp.float32),
            pltpu.VMEM((DBLK, D), jnp.float32),
            pltpu.VMEM_SHARED((CHUNK + 1, D), jnp.float32),
        ],
    )(features, lab3, centers)


def kernel(features, labels, centers):
    return _run(features, labels, centers)


# speedup vs baseline: 1.2620x; 1.2620x over previous
import jax
import jax.numpy as jnp
from jax import lax
from jax.experimental import pallas as pl
from jax.experimental.pallas import tpu as pltpu
from jax.experimental.pallas import tpu_sc as plsc

N_CENTER = 100000
D = 64
B = 16384
ALPHA = 0.9
A1 = ALPHA - 1.0

NC = 2
NS = 16
BT = B // NS
HALF = N_CENTER // NC
NCHUNK = 2
CHUNK = HALF // NCHUNK
TROWS = 1568
BLK = 64
NJ = BT // BLK
NDB = TROWS // BLK + 1


def _zero_fill(buf):
    def fill(i, _):
        buf[i // 4, pl.ds((i % 4) * 16, 16)] = jnp.zeros((16,), jnp.float32)
        return 0
    lax.fori_loop(0, BLK * 4, fill, 0)


def _body(feat_hbm, lab_hbm, ctr_hbm, out_hbm,
          bufa, bufb, bufc, bufd, lab_v, idx_v,
          sem_a, sem_b, sem_c, sem_d, sem_oa, sem_ob,
          acc_sh):
    c = lax.axis_index("c")
    s = lax.axis_index("s")

    pltpu.sync_copy(lab_hbm.at[s], lab_v)

    gath = [bufa, bufb]
    feat = [bufc, bufd]
    gsem = [sem_a, sem_b]
    fsem = [sem_c, sem_d]
    osem = [sem_oa, sem_ob]

    tstart = jnp.minimum(s * TROWS, CHUNK - TROWS)

    def dense_rb(b):
        return tstart + min(b * BLK, TROWS - BLK)

    for chunk in range(NCHUNK):
        base = c * HALF + chunk * CHUNK

        _zero_fill(bufa)
        zd = [pltpu.async_copy(bufa, acc_sh.at[pl.ds(dense_rb(b), BLK)],
                               sem_oa) for b in range(NDB)]
        for d in zd:
            d.wait()
        plsc.subcore_barrier()

        scat = [None, None]

        def issue_fetch(j):
            sl = j % 2
            g = pltpu.async_copy(ctr_hbm.at[lab_v.at[j]], gath[sl], gsem[sl])
            f = pltpu.async_copy(
                feat_hbm.at[pl.ds(s * BT + j * BLK, BLK)], feat[sl], fsem[sl])
            return g, f

        pend = issue_fetch(0)
        for j in range(NJ):
            sl = j % 2
            if j + 1 < NJ:
                if scat[(j + 1) % 2] is not None:
                    scat[(j + 1) % 2].wait()
                    scat[(j + 1) % 2] = None
                nxt = issue_fetch(j + 1)
            pend[0].wait()
            pend[1].wait()
            if j + 1 < NJ:
                pend2 = nxt

            def mkdiff(i, _):
                r = i // 4
                gg = (i % 4) * 16
                gath[sl][r, pl.ds(gg, 16)] = A1 * (
                    gath[sl][r, pl.ds(gg, 16)] - feat[sl][r, pl.ds(gg, 16)])
                return 0
            lax.fori_loop(0, BLK * 4, mkdiff, 0, unroll=8)

            for k in range(BLK // 16):
                v = lab_v[j, pl.ds(k * 16, 16)]
                rel = v - base
                inb = (rel >= 0) & (rel < CHUNK)
                idx_v[j, pl.ds(k * 16, 16)] = jnp.where(inb, rel, CHUNK)

            scat[sl] = pltpu.async_copy(
                gath[sl], acc_sh.at[idx_v.at[j]], osem[sl], add=True)
            if j + 1 < NJ:
                pend = pend2
        for d in scat:
            if d is not None:
                d.wait()
        plsc.subcore_barrier()

        owr = [None, None]

        def issue_dense(b):
            sl = b % 2
            rb = dense_rb(b)
            g = pltpu.async_copy(ctr_hbm.at[pl.ds(base + rb, BLK)],
                                 gath[sl], gsem[sl])
            f = pltpu.async_copy(acc_sh.at[pl.ds(rb, BLK)], feat[sl], fsem[sl])
            return g, f

        pend = issue_dense(0)
        for b in range(NDB):
            sl = b % 2
            if b + 1 < NDB:
                if owr[(b + 1) % 2] is not None:
                    owr[(b + 1) % 2].wait()
                    owr[(b + 1) % 2] = None
                nxt = issue_dense(b + 1)
            pend[0].wait()
            pend[1].wait()
            if b + 1 < NDB:
                pend2 = nxt

            def combine(i, _):
                r = i // 4
                gg = (i % 4) * 16
                gath[sl][r, pl.ds(gg, 16)] = (
                    gath[sl][r, pl.ds(gg, 16)] + feat[sl][r, pl.ds(gg, 16)])
                return 0
            lax.fori_loop(0, BLK * 4, combine, 0, unroll=8)

            owr[sl] = pltpu.async_copy(
                gath[sl], out_hbm.at[pl.ds(base + dense_rb(b), BLK)], osem[sl])
            if b + 1 < NDB:
                pend = pend2
        for d in owr:
            if d is not None:
                d.wait()

        if chunk + 1 < NCHUNK:
            plsc.subcore_barrier()


@jax.jit
def _run(features, labels, centers):
    mesh = plsc.VectorSubcoreMesh(core_axis_name="c", subcore_axis_name="s")
    lab3 = labels.reshape(NS, NJ, BLK)
    return pl.kernel(
        _body,
        out_type=jax.ShapeDtypeStruct((N_CENTER, D), jnp.float32),
        mesh=mesh,
        compiler_params=pltpu.CompilerParams(use_tc_tiling_on_sc=False),
        scratch_types=[
            pltpu.VMEM((BLK, D), jnp.float32),
            pltpu.VMEM((BLK, D), jnp.float32),
            pltpu.VMEM((BLK, D), jnp.float32),
            pltpu.VMEM((BLK, D), jnp.float32),
            pltpu.VMEM((NJ, BLK), jnp.int32),
            pltpu.VMEM((NJ, BLK), jnp.int32),
            pltpu.SemaphoreType.DMA,
            pltpu.SemaphoreType.DMA,
            pltpu.SemaphoreType.DMA,
            pltpu.SemaphoreType.DMA,
            pltpu.SemaphoreType.DMA,
            pltpu.SemaphoreType.DMA,
            pltpu.VMEM_SHARED((CHUNK + 1, D), jnp.float32),
        ],
    )(features, lab3, centers)


def kernel(features, labels, centers):
    return _run(features, labels, centers)
